# baseline (speedup 1.0000x reference)
import functools

import jax
import jax.numpy as jnp
from jax import lax
from jax.experimental import pallas as pl
from jax.experimental.pallas import tpu as pltpu
from jax.experimental.pallas import tpu_sc as plsc

_K = 16
_F = 1024
_MAXF = 1024.0
_L = 16
_NWORK = 32
_NBUF = 4
_NSL = _F // _L


_CP = (6.5286584, -25.9676, 60.167633, -85.45014, 64.93912, -19.739204, 1.0)


def _cos_2pi(u):
    v = u - jnp.round(u)
    z = v * v
    p = _CP[0] * z + _CP[1]
    for c in _CP[2:]:
        p = p * z + c
    return p


def _sc_topk_body(w_hbm, vals_hbm, freqs_hbm, buf, vacc, facc, sems, *,
                  base, rpw):
    wid = lax.axis_index("s") * 2 + lax.axis_index("c")
    row0 = base + wid * rpw
    iota = lax.iota(jnp.int32, _L)

    def _copy(b, row):
        return pltpu.make_async_copy(
            w_hbm.at[pl.ds(row0 + row, 1)], buf.at[pl.ds(b, 1)], sems.at[b])

    for b in range(_NBUF):
        _copy(b, b).start()

    def _merge_sorted(tv, ti, sv, si):
        rv = lax.rev(sv, (0,))
        ri = lax.rev(si, (0,))
        take = rv > tv
        tv = jnp.where(take, rv, tv)
        ti = jnp.where(take, ri, ti)
        kv = plsc.sort_key_val(tv, ti)
        return kv[0], kv[1]

    def _process(b, row):
        def _leaf(i):
            a = jnp.abs(buf[b, pl.ds(i * _L, _L)])
            kv = plsc.sort_key_val(a, iota + (i * _L))
            return kv[0], kv[1]

        def _tree(lo, n):
            if n == 1:
                return _leaf(lo)
            vlo, ilo = _tree(lo, n // 2)
            vhi, ihi = _tree(lo + n // 2, n - n // 2)
            return _merge_sorted(vlo, ilo, vhi, ihi)

        tv, ti = _tree(0, _NSL)
        signed = plsc.load_gather(buf, [jnp.full((_L,), b, jnp.int32), ti])
        base = row * _L
        plsc.store_scatter(vacc, [base + iota], signed)
        plsc.store_scatter(facc, [base + iota],
                           (_MAXF / _F) * ti.astype(jnp.float32))

    nsteps = (rpw + _NBUF - 1) // _NBUF

    def _step(s, _):
        for b in range(_NBUF):
            row = s * _NBUF + b

            @pl.when(row < rpw)
            def _do():
                _copy(b, row).wait()
                _process(b, row)

                @pl.when(row + _NBUF < rpw)
                def _next():
                    _copy(b, row + _NBUF).start()

        return 0

    lax.fori_loop(0, nsteps, _step, 0)
    out0 = wid * (rpw * _K)
    pltpu.sync_copy(vacc, vals_hbm.at[pl.ds(out0, rpw * _K)])
    pltpu.sync_copy(facc, freqs_hbm.at[pl.ds(out0, rpw * _K)])


def _render_body(xt_ref, colors_ref, pos_ref, scales_ref, rot_ref,
                 vals_ref, freqs_ref, out_ref):
    xx = xt_ref[0:1, :]
    xy = xt_ref[1:2, :]
    px = pos_ref[:, 0:1]
    py = pos_ref[:, 1:2]
    relx = xx - px
    rely = xy - py
    rot = rot_ref[:, 0:1]
    c = jnp.cos(rot)
    s = jnp.sin(rot)
    tx = c * relx + s * rely
    ty = -s * relx + c * rely
    sx = scales_ref[:, 0:1]
    sy = scales_ref[:, 1:2]
    env = jnp.exp(-0.5 * ((tx * sx) ** 2 + (ty * sy) ** 2))
    vals = vals_ref[...]
    freqs = freqs_ref[...]
    wave_x = jnp.zeros_like(tx)
    wave_y = jnp.zeros_like(ty)
    for k in range(_K):
        wave_x = wave_x + vals[:, k:k + 1] * _cos_2pi(freqs[:, k:k + 1] * tx)
        ky = _K + k
        wave_y = wave_y + vals[:, ky:ky + 1] * _cos_2pi(freqs[:, ky:ky + 1] * ty)
    w = env * wave_x * wave_y
    col = colors_ref[...]
    part = jnp.concatenate(
        [jnp.sum(w * col[:, c0:c0 + 1], axis=0, keepdims=True) for c0 in range(3)],
        axis=0)

    @pl.when(pl.program_id(0) == 0)
    def _init():
        out_ref[...] = jnp.zeros_like(out_ref)

    out_ref[...] += part


def kernel(x, gaussian_colors, gaussian_positions, gaussian_scales,
           gaussian_rotations, wave_coefficients):
    G = wave_coefficients.shape[0]
    N = x.shape[0]
    wave2 = wave_coefficients.reshape(2 * G, _F)
    xt = x.T
    Gb = 200

    def _sc_call(g0, gn):
        nrows = 2 * gn
        rpw = nrows // _NWORK
        body = functools.partial(_sc_topk_body, base=2 * g0, rpw=rpw)
        call = pl.kernel(
            body,
            out_type=[jax.ShapeDtypeStruct((nrows * _K,), jnp.float32),
                      jax.ShapeDtypeStruct((nrows * _K,), jnp.float32)],
            mesh=plsc.VectorSubcoreMesh(core_axis_name="c",
                                        subcore_axis_name="s"),
            compiler_params=pltpu.CompilerParams(needs_layout_passes=False),
            scratch_types=[
                pltpu.VMEM((_NBUF, _F), jnp.float32),
                pltpu.VMEM((rpw * _K,), jnp.float32),
                pltpu.VMEM((rpw * _K,), jnp.float32),
                pltpu.SemaphoreType.DMA((_NBUF,)),
            ],
        )
        return call(wave2)

    def _render_call(g0, gn, vals, freqs):
        vals2 = vals.reshape(gn, 2 * _K)
        freqs2 = freqs.reshape(gn, 2 * _K)
        return pl.pallas_call(
            _render_body,
            grid=(gn // Gb,),
            in_specs=[
                pl.BlockSpec((2, N), lambda i: (0, 0)),
                pl.BlockSpec((Gb, 3), lambda i: (i, 0)),
                pl.BlockSpec((Gb, 2), lambda i: (i, 0)),
                pl.BlockSpec((Gb, 2), lambda i: (i, 0)),
                pl.BlockSpec((Gb, 1), lambda i: (i, 0)),
                pl.BlockSpec((Gb, 2 * _K), lambda i: (i, 0)),
                pl.BlockSpec((Gb, 2 * _K), lambda i: (i, 0)),
            ],
            out_specs=pl.BlockSpec((3, N), lambda i: (0, 0)),
            out_shape=jax.ShapeDtypeStruct((3, N), jnp.float32),
        )(xt, gaussian_colors[g0:g0 + gn], gaussian_positions[g0:g0 + gn],
          gaussian_scales[g0:g0 + gn], gaussian_rotations[g0:g0 + gn],
          vals2, freqs2)

    vals, freqs = _sc_call(0, G)
    out_t = _render_call(0, G, vals, freqs)
    return out_t.T


# speedup vs baseline: 1.1632x; 1.1632x over previous
import functools

import jax
import jax.numpy as jnp
from jax import lax
from jax.experimental import pallas as pl
from jax.experimental.pallas import tpu as pltpu
from jax.experimental.pallas import tpu_sc as plsc

_K = 16
_F = 1024
_MAXF = 1024.0
_L = 16
_NWORK = 32
_NBUF = 2
_NSL = _F // _L


_CP = (6.5286584, -25.9676, 60.167633, -85.45014, 64.93912, -19.739204, 1.0)


def _cos_2pi(u):
    v = u - jnp.round(u)
    z = v * v
    p = _CP[0] * z + _CP[1]
    for c in _CP[2:]:
        p = p * z + c
    return p


def _sc_topk_body(w_hbm, vals_hbm, freqs_hbm, buf, vacc, facc, sems, *,
                  base, rpw):
    wid = lax.axis_index("s") * 2 + lax.axis_index("c")
    row0 = base + wid * rpw
    iota = lax.iota(jnp.int32, _L)

    def _copy(b, row):
        return pltpu.make_async_copy(
            w_hbm.at[pl.ds(row0 + row, 1)], buf.at[pl.ds(b, 1)], sems.at[b])

    for b in range(_NBUF):
        _copy(b, b).start()

    def _merge_sorted(tv, ti, sv, si):
        rv = lax.rev(sv, (0,))
        ri = lax.rev(si, (0,))
        take = rv > tv
        tv = jnp.where(take, rv, tv)
        ti = jnp.where(take, ri, ti)
        kv = plsc.sort_key_val(tv, ti)
        return kv[0], kv[1]

    def _process(b, row):
        def _leaf(i):
            a = jnp.abs(buf[b, pl.ds(i * _L, _L)])
            kv = plsc.sort_key_val(a, iota + (i * _L))
            return kv[0], kv[1]

        def _tree(lo, n):
            if n == 1:
                return _leaf(lo)
            vlo, ilo = _tree(lo, n // 2)
            vhi, ihi = _tree(lo + n // 2, n - n // 2)
            return _merge_sorted(vlo, ilo, vhi, ihi)

        tv, ti = _tree(0, _NSL)
        signed = plsc.load_gather(buf, [jnp.full((_L,), b, jnp.int32), ti])
        base = row * _L
        plsc.store_scatter(vacc, [base + iota], signed)
        plsc.store_scatter(facc, [base + iota],
                           (_MAXF / _F) * ti.astype(jnp.float32))

    nsteps = (rpw + _NBUF - 1) // _NBUF

    def _step(s, _):
        for b in range(_NBUF):
            row = s * _NBUF + b

            @pl.when(row < rpw)
            def _do():
                _copy(b, row).wait()
                _process(b, row)

                @pl.when(row + _NBUF < rpw)
                def _next():
                    _copy(b, row + _NBUF).start()

        return 0

    lax.fori_loop(0, nsteps, _step, 0)
    out0 = wid * (rpw * _K)
    pltpu.sync_copy(vacc, vals_hbm.at[pl.ds(out0, rpw * _K)])
    pltpu.sync_copy(facc, freqs_hbm.at[pl.ds(out0, rpw * _K)])


def _render_body(xt_ref, colors_ref, pos_ref, scales_ref, rot_ref,
                 vals_ref, freqs_ref, out_ref):
    xx = xt_ref[0:1, :]
    xy = xt_ref[1:2, :]
    px = pos_ref[:, 0:1]
    py = pos_ref[:, 1:2]
    relx = xx - px
    rely = xy - py
    rot = rot_ref[:, 0:1]
    c = jnp.cos(rot)
    s = jnp.sin(rot)
    tx = c * relx + s * rely
    ty = -s * relx + c * rely
    sx = scales_ref[:, 0:1]
    sy = scales_ref[:, 1:2]
    env = jnp.exp(-0.5 * ((tx * sx) ** 2 + (ty * sy) ** 2))
    vals = vals_ref[...]
    freqs = freqs_ref[...]
    wave_x = jnp.zeros_like(tx)
    wave_y = jnp.zeros_like(ty)
    for k in range(_K):
        wave_x = wave_x + vals[:, k:k + 1] * _cos_2pi(freqs[:, k:k + 1] * tx)
        ky = _K + k
        wave_y = wave_y + vals[:, ky:ky + 1] * _cos_2pi(freqs[:, ky:ky + 1] * ty)
    w = env * wave_x * wave_y
    col = colors_ref[...]
    part = jnp.concatenate(
        [jnp.sum(w * col[:, c0:c0 + 1], axis=0, keepdims=True) for c0 in range(3)],
        axis=0)

    @pl.when(pl.program_id(0) == 0)
    def _init():
        out_ref[...] = jnp.zeros_like(out_ref)

    out_ref[...] += part


def kernel(x, gaussian_colors, gaussian_positions, gaussian_scales,
           gaussian_rotations, wave_coefficients):
    G = wave_coefficients.shape[0]
    N = x.shape[0]
    wave2 = wave_coefficients.reshape(2 * G, _F)
    xt = x.T
    Gb = 200

    def _sc_call(g0, gn):
        nrows = 2 * gn
        rpw = nrows // _NWORK
        body = functools.partial(_sc_topk_body, base=2 * g0, rpw=rpw)
        call = pl.kernel(
            body,
            out_type=[jax.ShapeDtypeStruct((nrows * _K,), jnp.float32),
                      jax.ShapeDtypeStruct((nrows * _K,), jnp.float32)],
            mesh=plsc.VectorSubcoreMesh(core_axis_name="c",
                                        subcore_axis_name="s"),
            compiler_params=pltpu.CompilerParams(needs_layout_passes=False),
            scratch_types=[
                pltpu.VMEM((_NBUF, _F), jnp.float32),
                pltpu.VMEM((rpw * _K,), jnp.float32),
                pltpu.VMEM((rpw * _K,), jnp.float32),
                pltpu.SemaphoreType.DMA((_NBUF,)),
            ],
        )
        return call(wave2)

    def _render_call(g0, gn, vals, freqs):
        vals2 = vals.reshape(gn, 2 * _K)
        freqs2 = freqs.reshape(gn, 2 * _K)
        return pl.pallas_call(
            _render_body,
            grid=(gn // Gb,),
            in_specs=[
                pl.BlockSpec((2, N), lambda i: (0, 0)),
                pl.BlockSpec((Gb, 3), lambda i: (i, 0)),
                pl.BlockSpec((Gb, 2), lambda i: (i, 0)),
                pl.BlockSpec((Gb, 2), lambda i: (i, 0)),
                pl.BlockSpec((Gb, 1), lambda i: (i, 0)),
                pl.BlockSpec((Gb, 2 * _K), lambda i: (i, 0)),
                pl.BlockSpec((Gb, 2 * _K), lambda i: (i, 0)),
            ],
            out_specs=pl.BlockSpec((3, N), lambda i: (0, 0)),
            out_shape=jax.ShapeDtypeStruct((3, N), jnp.float32),
        )(xt, gaussian_colors[g0:g0 + gn], gaussian_positions[g0:g0 + gn],
          gaussian_scales[g0:g0 + gn], gaussian_rotations[g0:g0 + gn],
          vals2, freqs2)

    vals, freqs = _sc_call(0, G)
    out_t = _render_call(0, G, vals, freqs)
    return out_t.T


# speedup vs baseline: 1.2106x; 1.0407x over previous
import functools

import jax
import jax.numpy as jnp
from jax import lax
from jax.experimental import pallas as pl
from jax.experimental.pallas import tpu as pltpu
from jax.experimental.pallas import tpu_sc as plsc

_K = 16
_F = 1024
_MAXF = 1024.0
_L = 16
_NWORK = 32
_NBUF = 2
_NSL = _F // _L


_CP = (-21.071106, 58.790497, -85.27162, 64.92866, -19.738981, 0.9999992)


def _cos_2pi(u):
    v = u - jnp.round(u)
    z = v * v
    p = _CP[0] * z + _CP[1]
    for c in _CP[2:]:
        p = p * z + c
    return p


def _sc_topk_body(w_hbm, vals_hbm, freqs_hbm, buf, vacc, facc, sems, *,
                  base, rpw):
    wid = lax.axis_index("s") * 2 + lax.axis_index("c")
    row0 = base + wid * rpw
    iota = lax.iota(jnp.int32, _L)

    def _copy(b, row):
        return pltpu.make_async_copy(
            w_hbm.at[pl.ds(row0 + row, 1)], buf.at[pl.ds(b, 1)], sems.at[b])

    for b in range(_NBUF):
        _copy(b, b).start()

    def _merge_sorted(tv, ti, sv, si):
        rv = lax.rev(sv, (0,))
        ri = lax.rev(si, (0,))
        take = rv > tv
        tv = jnp.where(take, rv, tv)
        ti = jnp.where(take, ri, ti)
        kv = plsc.sort_key_val(tv, ti)
        return kv[0], kv[1]

    def _process(b, row):
        def _leaf(i):
            a = jnp.abs(buf[b, pl.ds(i * _L, _L)])
            kv = plsc.sort_key_val(a, iota + (i * _L))
            return kv[0], kv[1]

        def _tree(lo, n):
            if n == 1:
                return _leaf(lo)
            vlo, ilo = _tree(lo, n // 2)
            vhi, ihi = _tree(lo + n // 2, n - n // 2)
            return _merge_sorted(vlo, ilo, vhi, ihi)

        tv, ti = _tree(0, _NSL)
        signed = plsc.load_gather(buf, [jnp.full((_L,), b, jnp.int32), ti])
        base = row * _L
        plsc.store_scatter(vacc, [base + iota], signed)
        plsc.store_scatter(facc, [base + iota],
                           (_MAXF / _F) * ti.astype(jnp.float32))

    nsteps = (rpw + _NBUF - 1) // _NBUF

    def _step(s, _):
        for b in range(_NBUF):
            row = s * _NBUF + b

            @pl.when(row < rpw)
            def _do():
                _copy(b, row).wait()
                _process(b, row)

                @pl.when(row + _NBUF < rpw)
                def _next():
                    _copy(b, row + _NBUF).start()

        return 0

    lax.fori_loop(0, nsteps, _step, 0)
    out0 = wid * (rpw * _K)
    pltpu.sync_copy(vacc, vals_hbm.at[pl.ds(out0, rpw * _K)])
    pltpu.sync_copy(facc, freqs_hbm.at[pl.ds(out0, rpw * _K)])


def _render_body(xt_ref, colors_ref, pos_ref, scales_ref, rot_ref,
                 vals_ref, freqs_ref, out_ref):
    xx = xt_ref[0:1, :]
    xy = xt_ref[1:2, :]
    px = pos_ref[:, 0:1]
    py = pos_ref[:, 1:2]
    relx = xx - px
    rely = xy - py
    rot = rot_ref[:, 0:1]
    c = jnp.cos(rot)
    s = jnp.sin(rot)
    tx = c * relx + s * rely
    ty = -s * relx + c * rely
    sx = scales_ref[:, 0:1]
    sy = scales_ref[:, 1:2]
    env = jnp.exp(-0.5 * ((tx * sx) ** 2 + (ty * sy) ** 2))
    vals = vals_ref[...]
    freqs = freqs_ref[...]
    wave_x = jnp.zeros_like(tx)
    wave_y = jnp.zeros_like(ty)
    for k in range(_K):
        wave_x = wave_x + vals[:, k:k + 1] * _cos_2pi(freqs[:, k:k + 1] * tx)
        ky = _K + k
        wave_y = wave_y + vals[:, ky:ky + 1] * _cos_2pi(freqs[:, ky:ky + 1] * ty)
    w = env * wave_x * wave_y
    col = colors_ref[...]
    part = jnp.concatenate(
        [jnp.sum(w * col[:, c0:c0 + 1], axis=0, keepdims=True) for c0 in range(3)],
        axis=0)

    @pl.when(pl.program_id(0) == 0)
    def _init():
        out_ref[...] = jnp.zeros_like(out_ref)

    out_ref[...] += part


def kernel(x, gaussian_colors, gaussian_positions, gaussian_scales,
           gaussian_rotations, wave_coefficients):
    G = wave_coefficients.shape[0]
    N = x.shape[0]
    wave2 = wave_coefficients.reshape(2 * G, _F)
    xt = x.T
    Gb = 200

    def _sc_call(g0, gn):
        nrows = 2 * gn
        rpw = nrows // _NWORK
        body = functools.partial(_sc_topk_body, base=2 * g0, rpw=rpw)
        call = pl.kernel(
            body,
            out_type=[jax.ShapeDtypeStruct((nrows * _K,), jnp.float32),
                      jax.ShapeDtypeStruct((nrows * _K,), jnp.float32)],
            mesh=plsc.VectorSubcoreMesh(core_axis_name="c",
                                        subcore_axis_name="s"),
            compiler_params=pltpu.CompilerParams(needs_layout_passes=False),
            scratch_types=[
                pltpu.VMEM((_NBUF, _F), jnp.float32),
                pltpu.VMEM((rpw * _K,), jnp.float32),
                pltpu.VMEM((rpw * _K,), jnp.float32),
                pltpu.SemaphoreType.DMA((_NBUF,)),
            ],
        )
        return call(wave2)

    def _render_call(g0, gn, vals, freqs):
        vals2 = vals.reshape(gn, 2 * _K)
        freqs2 = freqs.reshape(gn, 2 * _K)
        return pl.pallas_call(
            _render_body,
            grid=(gn // Gb,),
            in_specs=[
                pl.BlockSpec((2, N), lambda i: (0, 0)),
                pl.BlockSpec((Gb, 3), lambda i: (i, 0)),
                pl.BlockSpec((Gb, 2), lambda i: (i, 0)),
                pl.BlockSpec((Gb, 2), lambda i: (i, 0)),
                pl.BlockSpec((Gb, 1), lambda i: (i, 0)),
                pl.BlockSpec((Gb, 2 * _K), lambda i: (i, 0)),
                pl.BlockSpec((Gb, 2 * _K), lambda i: (i, 0)),
            ],
            out_specs=pl.BlockSpec((3, N), lambda i: (0, 0)),
            out_shape=jax.ShapeDtypeStruct((3, N), jnp.float32),
        )(xt, gaussian_colors[g0:g0 + gn], gaussian_positions[g0:g0 + gn],
          gaussian_scales[g0:g0 + gn], gaussian_rotations[g0:g0 + gn],
          vals2, freqs2)

    vals, freqs = _sc_call(0, G)
    out_t = _render_call(0, G, vals, freqs)
    return out_t.T
